# Initial kernel scaffold; baseline (speedup 1.0000x reference)
import functools

import jax
import jax.numpy as jnp
from jax import lax
from jax.experimental import pallas as pl
from jax.experimental.pallas import tpu as pltpu
from jax.experimental.pallas import tpu_sc as plsc

B = 16384
WINDOW = 5
NEG = 5
MWL = 5
SIZE = 64
NCTX = 2 * WINDOW
NCHR = 2 * WINDOW * MWL

NC = 2
NS = 16
NW = NC * NS
RPW = B // NW
C = 16
NCHUNK = RPW // C


def _sc_body(ctx_idx, lens, tar, neg, char_idx, cn,
             emb0, emb1, emb0c,
             pos_out, neg_out,
             widx_v, cidx_v, nidx_v, tidx_v, lens_v, cn_v, hl_v, hc_v,
             wrows, crows, nrows, trows, posc, negc, sem):
    wid = lax.axis_index("s") * NC + lax.axis_index("c")
    base = wid * RPW

    def chunk(g, carry):
        r0 = base + g * C

        pltpu.sync_copy(ctx_idx.at[pl.ds(r0 * NCTX, C * NCTX)], widx_v)
        pltpu.sync_copy(char_idx.at[pl.ds(r0 * NCHR, C * NCHR)], cidx_v)
        pltpu.sync_copy(neg.at[pl.ds(r0 * NEG, C * NEG)], nidx_v)
        pltpu.sync_copy(tar.at[pl.ds(r0, C)], tidx_v)
        pltpu.sync_copy(lens.at[pl.ds(r0, C)], lens_v)
        pltpu.sync_copy(cn.at[pl.ds(r0, C)], cn_v)

        hs = []
        hs.append(pltpu.async_copy(
            emb0.at[widx_v.at[pl.ds(0, 128)]], wrows.at[pl.ds(0, 128)], sem))
        hs.append(pltpu.async_copy(
            emb0.at[widx_v.at[pl.ds(128, 32)]], wrows.at[pl.ds(128, 32)], sem))
        for t in range(6):
            hs.append(pltpu.async_copy(
                emb0c.at[cidx_v.at[pl.ds(t * 128, 128)]],
                crows.at[pl.ds(t * 128, 128)], sem))
        hs.append(pltpu.async_copy(
            emb0c.at[cidx_v.at[pl.ds(768, 32)]], crows.at[pl.ds(768, 32)], sem))
        hs.append(pltpu.async_copy(emb1.at[nidx_v], nrows, sem))
        hs.append(pltpu.async_copy(emb1.at[tidx_v], trows, sem))
        for h in hs:
            h.wait()

        hl_v[...] = 0.5 / lens_v[...]
        hc_v[...] = 0.5 / cn_v[...]

        def row(r, carry2):
            ridx = jnp.full((16,), r, jnp.int32)
            hl = plsc.load_gather(hl_v, [ridx])
            hc = plsc.load_gather(hc_v, [ridx])
            avg = []
            for d in range(4):
                sl = pl.ds(d * 16, 16)
                ws = wrows[r * NCTX, sl]
                for j in range(1, NCTX):
                    ws = ws + wrows[r * NCTX + j, sl]
                cs = crows[r * NCHR, sl]
                for j in range(1, NCHR):
                    cs = cs + crows[r * NCHR + j, sl]
                avg.append(ws * hl + cs * hc)
            pv = avg[0] * trows[r, pl.ds(0, 16)]
            for d in range(1, 4):
                pv = pv + avg[d] * trows[r, pl.ds(d * 16, 16)]
            posc[r] = jnp.sum(pv)
            for n in range(NEG):
                nv = avg[0] * nrows[r * NEG + n, pl.ds(0, 16)]
                for d in range(1, 4):
                    nv = nv + avg[d] * nrows[r * NEG + n, pl.ds(d * 16, 16)]
                negc[r * NEG + n] = jnp.sum(nv)
            return carry2

        lax.fori_loop(0, C, row, 0)
        pltpu.sync_copy(posc, pos_out.at[pl.ds(r0, C)])
        pltpu.sync_copy(negc, neg_out.at[pl.ds(r0 * NEG, C * NEG)])
        return carry

    lax.fori_loop(0, NCHUNK, chunk, 0)


def _sc_ips(ctx_idx, lens, tar, neg, char_idx, cn, emb0, emb1, emb0c):
    mesh = plsc.VectorSubcoreMesh(core_axis_name="c", subcore_axis_name="s")
    return pl.kernel(
        _sc_body,
        mesh=mesh,
        out_type=[
            jax.ShapeDtypeStruct((B,), jnp.float32),
            jax.ShapeDtypeStruct((B * NEG,), jnp.float32),
        ],
        scratch_types=[
            pltpu.VMEM((C * NCTX,), jnp.int32),
            pltpu.VMEM((C * NCHR,), jnp.int32),
            pltpu.VMEM((C * NEG,), jnp.int32),
            pltpu.VMEM((C,), jnp.int32),
            pltpu.VMEM((C,), jnp.float32),
            pltpu.VMEM((C,), jnp.float32),
            pltpu.VMEM((C,), jnp.float32),
            pltpu.VMEM((C,), jnp.float32),
            pltpu.VMEM((C * NCTX, SIZE), jnp.float32),
            pltpu.VMEM((C * NCHR, SIZE), jnp.float32),
            pltpu.VMEM((C * NEG, SIZE), jnp.float32),
            pltpu.VMEM((C, SIZE), jnp.float32),
            pltpu.VMEM((C,), jnp.float32),
            pltpu.VMEM((C * NEG,), jnp.float32),
            pltpu.SemaphoreType.DMA,
        ],
    )(ctx_idx, lens, tar, neg, char_idx, cn, emb0, emb1, emb0c)


def _tc_loss_body(pos_ref, neg_ref, mask_ref, out_ref):
    p = jnp.clip(pos_ref[...], -10.0, 10.0)
    pos_loss = jnp.sum(jnp.log(1.0 + jnp.exp(-p)))
    m = mask_ref[...]
    q = jnp.clip(neg_ref[...] * m, -10.0, 10.0)
    neg_loss = jnp.sum(jnp.log(1.0 + jnp.exp(q)) * m)
    out_ref[0, 0] = pos_loss + neg_loss


def _tc_loss(pos2d, neg2d, mask2d):
    return pl.pallas_call(
        _tc_loss_body,
        out_shape=jax.ShapeDtypeStruct((1, 1), jnp.float32),
        out_specs=pl.BlockSpec(memory_space=pltpu.SMEM),
    )(pos2d, neg2d, mask2d)


def kernel(word_data, char_data, emb0_w, emb1_w, emb0_char_w):
    wd = word_data.astype(jnp.int32)
    cd = char_data.astype(jnp.int32)
    ctx_idx = wd[:, 0:NCTX].reshape(-1)
    lens = wd[:, NCTX].astype(jnp.float32)
    tar = wd[:, NCTX + 1]
    neg = wd[:, NCTX + 2:NCTX + 2 + NEG].reshape(-1)
    mask = wd[:, NCTX + 2 + NEG:].astype(jnp.float32)
    char_idx = cd[:, 0:NCHR].reshape(-1)
    cn = cd[:, NCHR].astype(jnp.float32)

    pos_ips, neg_ips = _sc_ips(ctx_idx, lens, tar, neg, char_idx, cn,
                               emb0_w, emb1_w, emb0_char_w)
    loss = _tc_loss(pos_ips.reshape(128, 128),
                    neg_ips.reshape(B * NEG // 128, 128),
                    mask.reshape(-1).reshape(B * NEG // 128, 128))
    return loss[0, 0]


# speedup vs baseline: 9.6509x; 9.6509x over previous
import functools

import jax
import jax.numpy as jnp
from jax import lax
from jax.experimental import pallas as pl
from jax.experimental.pallas import tpu as pltpu
from jax.experimental.pallas import tpu_sc as plsc

B = 16384
WINDOW = 5
NEG = 5
MWL = 5
SIZE = 64
NCTX = 2 * WINDOW
NCHR = 2 * WINDOW * MWL

NC = 2
NS = 16
NW = NC * NS
RPW = B // NW
C = 16
NCHUNK = RPW // C


def _sc_body(ctx_idx, lens, tar, neg, char_idx, cn,
             emb0, emb1, emb0c,
             pos_out, neg_out,
             widx_v, cidx_v, nidx_v, tidx_v, lens_v, cn_v,
             wrows, crows, nrows, trows, posc, negc, sem):
    wid = lax.axis_index("s") * NC + lax.axis_index("c")
    base = wid * RPW

    def chunk(g, carry):
        r0 = base + g * C

        pltpu.sync_copy(ctx_idx.at[pl.ds(r0 * NCTX, C * NCTX)], widx_v)
        pltpu.sync_copy(char_idx.at[pl.ds(r0 * NCHR, C * NCHR)], cidx_v)
        pltpu.sync_copy(neg.at[pl.ds(r0 * NEG, C * NEG)], nidx_v)
        pltpu.sync_copy(tar.at[pl.ds(r0, C)], tidx_v)
        pltpu.sync_copy(lens.at[pl.ds(r0, C)], lens_v)
        pltpu.sync_copy(cn.at[pl.ds(r0, C)], cn_v)

        hs = []
        hs.append(pltpu.async_copy(
            emb0.at[widx_v.at[pl.ds(0, 128)]], wrows.at[pl.ds(0, 128)], sem))
        hs.append(pltpu.async_copy(
            emb0.at[widx_v.at[pl.ds(128, 32)]], wrows.at[pl.ds(128, 32)], sem))
        for t in range(6):
            hs.append(pltpu.async_copy(
                emb0c.at[cidx_v.at[pl.ds(t * 128, 128)]],
                crows.at[pl.ds(t * 128, 128)], sem))
        hs.append(pltpu.async_copy(
            emb0c.at[cidx_v.at[pl.ds(768, 32)]], crows.at[pl.ds(768, 32)], sem))
        hs.append(pltpu.async_copy(emb1.at[nidx_v], nrows, sem))
        hs.append(pltpu.async_copy(emb1.at[tidx_v], trows, sem))
        for h in hs:
            h.wait()

        hlv = 0.5 / lens_v[...]
        hcv = 0.5 / cn_v[...]

        lane = lax.broadcasted_iota(jnp.int32, (16,), 0)
        idx15 = jnp.full((16,), 15, jnp.int32)

        def lane_total(v):
            return plsc.cumsum(v).at[idx15].get(mode="promise_in_bounds")

        def row(r, carry2):
            accs = carry2
            ridx = jnp.full((16,), r, jnp.int32)
            hl = hlv.at[ridx].get(mode="promise_in_bounds")
            hc = hcv.at[ridx].get(mode="promise_in_bounds")
            avg = []
            for d in range(4):
                sl = pl.ds(d * 16, 16)
                ws = wrows[r * NCTX, sl]
                for j in range(1, NCTX):
                    ws = ws + wrows[r * NCTX + j, sl]
                cs = crows[r * NCHR, sl]
                for j in range(1, NCHR):
                    cs = cs + crows[r * NCHR + j, sl]
                avg.append(ws * hl + cs * hc)
            msk_r = lane == r
            pv = avg[0] * trows[r, pl.ds(0, 16)]
            for d in range(1, 4):
                pv = pv + avg[d] * trows[r, pl.ds(d * 16, 16)]
            out = [jnp.where(msk_r, lane_total(pv), accs[0])]
            for n in range(NEG):
                nv = avg[0] * nrows[r * NEG + n, pl.ds(0, 16)]
                for d in range(1, 4):
                    nv = nv + avg[d] * nrows[r * NEG + n, pl.ds(d * 16, 16)]
                out.append(jnp.where(msk_r, lane_total(nv), accs[n + 1]))
            return tuple(out)

        zero = jnp.zeros((16,), jnp.float32)
        accs = lax.fori_loop(0, C, row, (zero,) * (1 + NEG))
        posc[...] = accs[0]
        for n in range(NEG):
            negc[n] = accs[n + 1]
        pltpu.sync_copy(posc, pos_out.at[pl.ds(r0, C)])
        for n in range(NEG):
            pltpu.sync_copy(negc.at[n], neg_out.at[pl.ds(n * B + r0, C)])
        return carry

    lax.fori_loop(0, NCHUNK, chunk, 0)


def _sc_ips(ctx_idx, lens, tar, neg, char_idx, cn, emb0, emb1, emb0c):
    mesh = plsc.VectorSubcoreMesh(core_axis_name="c", subcore_axis_name="s")
    return pl.kernel(
        _sc_body,
        mesh=mesh,
        compiler_params=pltpu.CompilerParams(
            needs_layout_passes=False, use_tc_tiling_on_sc=False),
        out_type=[
            jax.ShapeDtypeStruct((B,), jnp.float32),
            jax.ShapeDtypeStruct((B * NEG,), jnp.float32),
        ],
        scratch_types=[
            pltpu.VMEM((C * NCTX,), jnp.int32),
            pltpu.VMEM((C * NCHR,), jnp.int32),
            pltpu.VMEM((C * NEG,), jnp.int32),
            pltpu.VMEM((C,), jnp.int32),
            pltpu.VMEM((C,), jnp.float32),
            pltpu.VMEM((C,), jnp.float32),
            pltpu.VMEM((C * NCTX, SIZE), jnp.float32),
            pltpu.VMEM((C * NCHR, SIZE), jnp.float32),
            pltpu.VMEM((C * NEG, SIZE), jnp.float32),
            pltpu.VMEM((C, SIZE), jnp.float32),
            pltpu.VMEM((C,), jnp.float32),
            pltpu.VMEM((NEG, C), jnp.float32),
            pltpu.SemaphoreType.DMA,
        ],
    )(ctx_idx, lens, tar, neg, char_idx, cn, emb0, emb1, emb0c)


def _tc_loss_body(pos_ref, neg_ref, mask_ref, out_ref):
    p = jnp.clip(pos_ref[...], -10.0, 10.0)
    pos_loss = jnp.sum(jnp.log(1.0 + jnp.exp(-p)))
    m = mask_ref[...]
    q = jnp.clip(neg_ref[...] * m, -10.0, 10.0)
    neg_loss = jnp.sum(jnp.log(1.0 + jnp.exp(q)) * m)
    out_ref[0, 0] = pos_loss + neg_loss


def _tc_loss(pos2d, neg2d, mask2d):
    return pl.pallas_call(
        _tc_loss_body,
        out_shape=jax.ShapeDtypeStruct((1, 1), jnp.float32),
        out_specs=pl.BlockSpec(memory_space=pltpu.SMEM),
    )(pos2d, neg2d, mask2d)


def kernel(word_data, char_data, emb0_w, emb1_w, emb0_char_w):
    wd = word_data.astype(jnp.int32)
    cd = char_data.astype(jnp.int32)
    ctx_idx = wd[:, 0:NCTX].reshape(-1)
    lens = wd[:, NCTX].astype(jnp.float32)
    tar = wd[:, NCTX + 1]
    neg = wd[:, NCTX + 2:NCTX + 2 + NEG].reshape(-1)
    mask = wd[:, NCTX + 2 + NEG:].astype(jnp.float32)
    char_idx = cd[:, 0:NCHR].reshape(-1)
    cn = cd[:, NCHR].astype(jnp.float32)

    pos_ips, neg_ips = _sc_ips(ctx_idx, lens, tar, neg, char_idx, cn,
                               emb0_w, emb1_w, emb0_char_w)
    loss = _tc_loss(pos_ips.reshape(128, 128),
                    neg_ips.reshape(B * NEG // 128, 128),
                    mask.T.reshape(B * NEG // 128, 128))
    return loss[0, 0]


# speedup vs baseline: 18.1355x; 1.8792x over previous
import functools

import jax
import jax.numpy as jnp
from jax import lax
from jax.experimental import pallas as pl
from jax.experimental.pallas import tpu as pltpu
from jax.experimental.pallas import tpu_sc as plsc

B = 16384
WINDOW = 5
NEG = 5
MWL = 5
SIZE = 64
NCTX = 2 * WINDOW
NCHR = 2 * WINDOW * MWL

NC = 2
NS = 16
NW = NC * NS
RPW = B // NW
C = 64
NCHUNK = RPW // C
NQ = C // 16


def _sc_body(ctx_idx, lens, tar, neg, char_idx, cn,
             emb0, emb1, emb0c,
             pos_out, neg_out,
             widx_v, cidx_v, nidx_v, tidx_v, lens_v, cn_v,
             wsum, csum, nrows, trows, posc, negc, sem):
    wid = lax.axis_index("s") * NC + lax.axis_index("c")
    base = wid * RPW

    pltpu.sync_copy(ctx_idx.at[:, pl.ds(base, RPW)], widx_v)
    pltpu.sync_copy(char_idx.at[:, pl.ds(base, RPW)], cidx_v)
    pltpu.sync_copy(neg.at[pl.ds(base * NEG, RPW * NEG)], nidx_v)
    pltpu.sync_copy(tar.at[pl.ds(base, RPW)], tidx_v)
    pltpu.sync_copy(lens.at[pl.ds(base, RPW)], lens_v)
    pltpu.sync_copy(cn.at[pl.ds(base, RPW)], cn_v)

    lane = lax.broadcasted_iota(jnp.int32, (16,), 0)
    idx15 = jnp.full((16,), 15, jnp.int32)
    zero16 = jnp.zeros((16,), jnp.float32)

    def lane_total(v):
        return plsc.cumsum(v).at[idx15].get(mode="promise_in_bounds")

    def chunk(g, carry):
        c0 = g * C

        def zrow(i, cz):
            for d in range(4):
                wsum[i, pl.ds(d * 16, 16)] = zero16
                csum[i, pl.ds(d * 16, 16)] = zero16
            return cz
        lax.fori_loop(0, C, zrow, 0)

        hs = []
        for j in range(NCTX):
            hs.append(pltpu.async_copy(
                emb0.at[widx_v.at[j, pl.ds(c0, C)]], wsum, sem, add=True))
        for j in range(NCHR):
            hs.append(pltpu.async_copy(
                emb0c.at[cidx_v.at[j, pl.ds(c0, C)]], csum, sem, add=True))
        for t in range(2):
            hs.append(pltpu.async_copy(
                emb1.at[nidx_v.at[pl.ds(c0 * NEG + t * 128, 128)]],
                nrows.at[pl.ds(t * 128, 128)], sem))
        hs.append(pltpu.async_copy(
            emb1.at[nidx_v.at[pl.ds(c0 * NEG + 256, 64)]],
            nrows.at[pl.ds(256, 64)], sem))
        hs.append(pltpu.async_copy(
            emb1.at[tidx_v.at[pl.ds(c0, C)]], trows, sem))
        for h in hs:
            h.wait()

        for q in range(NQ):
            hlv = 0.5 / lens_v[pl.ds(c0 + q * 16, 16)]
            hcv = 0.5 / cn_v[pl.ds(c0 + q * 16, 16)]

            def row(r2, carry2):
                accs = carry2
                r = q * 16 + r2
                ridx = jnp.full((16,), r2, jnp.int32)
                hl = hlv.at[ridx].get(mode="promise_in_bounds")
                hc = hcv.at[ridx].get(mode="promise_in_bounds")
                avg = []
                for d in range(4):
                    sl = pl.ds(d * 16, 16)
                    avg.append(wsum[r, sl] * hl + csum[r, sl] * hc)
                msk_r = lane == r2
                pv = avg[0] * trows[r, pl.ds(0, 16)]
                for d in range(1, 4):
                    pv = pv + avg[d] * trows[r, pl.ds(d * 16, 16)]
                out = [jnp.where(msk_r, lane_total(pv), accs[0])]
                for n in range(NEG):
                    nv = avg[0] * nrows[r * NEG + n, pl.ds(0, 16)]
                    for d in range(1, 4):
                        nv = nv + avg[d] * nrows[r * NEG + n, pl.ds(d * 16, 16)]
                    out.append(jnp.where(msk_r, lane_total(nv), accs[n + 1]))
                return tuple(out)

            accs = lax.fori_loop(0, 16, row, (zero16,) * (1 + NEG))
            posc[pl.ds(q * 16, 16)] = accs[0]
            for n in range(NEG):
                negc[n, pl.ds(q * 16, 16)] = accs[n + 1]

        pltpu.sync_copy(posc, pos_out.at[pl.ds(base + c0, C)])
        for n in range(NEG):
            pltpu.sync_copy(negc.at[n], neg_out.at[pl.ds(n * B + base + c0, C)])
        return carry

    lax.fori_loop(0, NCHUNK, chunk, 0)


def _sc_ips(ctx_idx, lens, tar, neg, char_idx, cn, emb0, emb1, emb0c):
    mesh = plsc.VectorSubcoreMesh(core_axis_name="c", subcore_axis_name="s")
    return pl.kernel(
        _sc_body,
        mesh=mesh,
        compiler_params=pltpu.CompilerParams(
            needs_layout_passes=False, use_tc_tiling_on_sc=False),
        out_type=[
            jax.ShapeDtypeStruct((B,), jnp.float32),
            jax.ShapeDtypeStruct((B * NEG,), jnp.float32),
        ],
        scratch_types=[
            pltpu.VMEM((NCTX, RPW), jnp.int32),
            pltpu.VMEM((NCHR, RPW), jnp.int32),
            pltpu.VMEM((RPW * NEG,), jnp.int32),
            pltpu.VMEM((RPW,), jnp.int32),
            pltpu.VMEM((RPW,), jnp.float32),
            pltpu.VMEM((RPW,), jnp.float32),
            pltpu.VMEM((C, SIZE), jnp.float32),
            pltpu.VMEM((C, SIZE), jnp.float32),
            pltpu.VMEM((C * NEG, SIZE), jnp.float32),
            pltpu.VMEM((C, SIZE), jnp.float32),
            pltpu.VMEM((C,), jnp.float32),
            pltpu.VMEM((NEG, C), jnp.float32),
            pltpu.SemaphoreType.DMA,
        ],
    )(ctx_idx, lens, tar, neg, char_idx, cn, emb0, emb1, emb0c)


def _tc_loss_body(pos_ref, neg_ref, mask_ref, out_ref):
    p = jnp.clip(pos_ref[...], -10.0, 10.0)
    pos_loss = jnp.sum(jnp.log(1.0 + jnp.exp(-p)))
    m = mask_ref[...]
    q = jnp.clip(neg_ref[...] * m, -10.0, 10.0)
    neg_loss = jnp.sum(jnp.log(1.0 + jnp.exp(q)) * m)
    out_ref[0, 0] = pos_loss + neg_loss


def _tc_loss(pos2d, neg2d, mask2d):
    return pl.pallas_call(
        _tc_loss_body,
        out_shape=jax.ShapeDtypeStruct((1, 1), jnp.float32),
        out_specs=pl.BlockSpec(memory_space=pltpu.SMEM),
    )(pos2d, neg2d, mask2d)


def kernel(word_data, char_data, emb0_w, emb1_w, emb0_char_w):
    wd = word_data.astype(jnp.int32)
    cd = char_data.astype(jnp.int32)
    ctx_idx = wd[:, 0:NCTX].T
    lens = wd[:, NCTX].astype(jnp.float32)
    tar = wd[:, NCTX + 1]
    neg = wd[:, NCTX + 2:NCTX + 2 + NEG].reshape(-1)
    mask = wd[:, NCTX + 2 + NEG:].astype(jnp.float32)
    char_idx = cd[:, 0:NCHR].T
    cn = cd[:, NCHR].astype(jnp.float32)

    pos_ips, neg_ips = _sc_ips(ctx_idx, lens, tar, neg, char_idx, cn,
                               emb0_w, emb1_w, emb0_char_w)
    loss = _tc_loss(pos_ips.reshape(128, 128),
                    neg_ips.reshape(B * NEG // 128, 128),
                    mask.T.reshape(B * NEG // 128, 128))
    return loss[0, 0]


# speedup vs baseline: 18.8353x; 1.0386x over previous
import functools

import jax
import jax.numpy as jnp
from jax import lax
from jax.experimental import pallas as pl
from jax.experimental.pallas import tpu as pltpu
from jax.experimental.pallas import tpu_sc as plsc

B = 16384
WINDOW = 5
NEG = 5
MWL = 5
SIZE = 64
NCTX = 2 * WINDOW
NCHR = 2 * WINDOW * MWL

NC = 2
NS = 16
NW = NC * NS
RPW = B // NW
C = 64
NCHUNK = RPW // C
NQ = C // 16


def _sc_body(ctx_idx, lens, tar, neg, char_idx, cn,
             emb0, emb1, emb0c,
             pos_out, neg_out,
             widx_v, cidx_v, nidx_v, tidx_v, lens_v, cn_v,
             wsum0, csum0, nrows0, trows0,
             wsum1, csum1, nrows1, trows1,
             posc, negc, sem0, sem1):
    wid = lax.axis_index("s") * NC + lax.axis_index("c")
    base = wid * RPW

    pltpu.sync_copy(ctx_idx.at[:, pl.ds(base, RPW)], widx_v)
    pltpu.sync_copy(char_idx.at[:, pl.ds(base, RPW)], cidx_v)
    pltpu.sync_copy(neg.at[pl.ds(base * NEG, RPW * NEG)], nidx_v)
    pltpu.sync_copy(tar.at[pl.ds(base, RPW)], tidx_v)
    pltpu.sync_copy(lens.at[pl.ds(base, RPW)], lens_v)
    pltpu.sync_copy(cn.at[pl.ds(base, RPW)], cn_v)

    lane = lax.broadcasted_iota(jnp.int32, (16,), 0)
    idx15 = jnp.full((16,), 15, jnp.int32)
    zero16 = jnp.zeros((16,), jnp.float32)

    def lane_total(v):
        return plsc.cumsum(v).at[idx15].get(mode="promise_in_bounds")

    def zero_bufs(wsum, csum):
        def zrow(i, cz):
            for d in range(4):
                wsum[i, pl.ds(d * 16, 16)] = zero16
                csum[i, pl.ds(d * 16, 16)] = zero16
            return cz
        lax.fori_loop(0, C, zrow, 0)

    def transfers(g, wsum, csum, nrows, trows, sem):
        c0 = g * C
        ts = []
        for j in range(NCTX):
            ts.append((emb0.at[widx_v.at[j, pl.ds(c0, C)]], wsum, sem, True))
        for j in range(NCHR):
            ts.append((emb0c.at[cidx_v.at[j, pl.ds(c0, C)]], csum, sem, True))
        for t in range(2):
            ts.append((emb1.at[nidx_v.at[pl.ds(c0 * NEG + t * 128, 128)]],
                       nrows.at[pl.ds(t * 128, 128)], sem, False))
        ts.append((emb1.at[nidx_v.at[pl.ds(c0 * NEG + 256, 64)]],
                   nrows.at[pl.ds(256, 64)], sem, False))
        ts.append((emb1.at[tidx_v.at[pl.ds(c0, C)]], trows, sem, False))
        return ts

    def fire(g, wsum, csum, nrows, trows, sem):
        for src, dst, s, add in transfers(g, wsum, csum, nrows, trows, sem):
            pltpu.async_copy(src, dst, s, add=add)

    def drain(g, wsum, csum, nrows, trows, sem):
        for src, dst, s, add in transfers(g, wsum, csum, nrows, trows, sem):
            pltpu.make_async_copy(src, dst, s).wait()

    def compute(g, wsum, csum, nrows, trows):
        c0 = g * C
        for q in range(NQ):
            hlv = 0.5 / lens_v[pl.ds(c0 + q * 16, 16)]
            hcv = 0.5 / cn_v[pl.ds(c0 + q * 16, 16)]

            def row(r2, carry2):
                accs = carry2
                r = q * 16 + r2
                ridx = jnp.full((16,), r2, jnp.int32)
                hl = hlv.at[ridx].get(mode="promise_in_bounds")
                hc = hcv.at[ridx].get(mode="promise_in_bounds")
                avg = []
                for d in range(4):
                    sl = pl.ds(d * 16, 16)
                    avg.append(wsum[r, sl] * hl + csum[r, sl] * hc)
                msk_r = lane == r2
                pv = avg[0] * trows[r, pl.ds(0, 16)]
                for d in range(1, 4):
                    pv = pv + avg[d] * trows[r, pl.ds(d * 16, 16)]
                out = [jnp.where(msk_r, lane_total(pv), accs[0])]
                for n in range(NEG):
                    nv = avg[0] * nrows[r * NEG + n, pl.ds(0, 16)]
                    for d in range(1, 4):
                        nv = nv + avg[d] * nrows[r * NEG + n, pl.ds(d * 16, 16)]
                    out.append(jnp.where(msk_r, lane_total(nv), accs[n + 1]))
                return tuple(out)

            accs = lax.fori_loop(0, 16, row, (zero16,) * (1 + NEG))
            posc[pl.ds(q * 16, 16)] = accs[0]
            for n in range(NEG):
                negc[n, pl.ds(q * 16, 16)] = accs[n + 1]

        pltpu.sync_copy(posc, pos_out.at[pl.ds(base + c0, C)])
        for n in range(NEG):
            pltpu.sync_copy(negc.at[n], neg_out.at[pl.ds(n * B + base + c0, C)])

    bufs0 = (wsum0, csum0, nrows0, trows0, sem0)
    bufs1 = (wsum1, csum1, nrows1, trows1, sem1)

    zero_bufs(wsum0, csum0)
    zero_bufs(wsum1, csum1)
    fire(0, *bufs0)
    fire(1, *bufs1)

    def body(k, carry):
        g0 = 2 * k
        for g, bufs in ((g0, bufs0), (g0 + 1, bufs1)):
            drain(g, *bufs)
            compute(g, *bufs[:4])
            zero_bufs(bufs[0], bufs[1])

            @pl.when(g + 2 < NCHUNK)
            def _():
                fire(g + 2, *bufs)
        return carry

    lax.fori_loop(0, NCHUNK // 2, body, 0)


def _sc_ips(ctx_idx, lens, tar, neg, char_idx, cn, emb0, emb1, emb0c):
    mesh = plsc.VectorSubcoreMesh(core_axis_name="c", subcore_axis_name="s")
    return pl.kernel(
        _sc_body,
        mesh=mesh,
        compiler_params=pltpu.CompilerParams(
            needs_layout_passes=False, use_tc_tiling_on_sc=False),
        out_type=[
            jax.ShapeDtypeStruct((B,), jnp.float32),
            jax.ShapeDtypeStruct((B * NEG,), jnp.float32),
        ],
        scratch_types=[
            pltpu.VMEM((NCTX, RPW), jnp.int32),
            pltpu.VMEM((NCHR, RPW), jnp.int32),
            pltpu.VMEM((RPW * NEG,), jnp.int32),
            pltpu.VMEM((RPW,), jnp.int32),
            pltpu.VMEM((RPW,), jnp.float32),
            pltpu.VMEM((RPW,), jnp.float32),
            pltpu.VMEM((C, SIZE), jnp.float32),
            pltpu.VMEM((C, SIZE), jnp.float32),
            pltpu.VMEM((C * NEG, SIZE), jnp.float32),
            pltpu.VMEM((C, SIZE), jnp.float32),
            pltpu.VMEM((C, SIZE), jnp.float32),
            pltpu.VMEM((C, SIZE), jnp.float32),
            pltpu.VMEM((C * NEG, SIZE), jnp.float32),
            pltpu.VMEM((C, SIZE), jnp.float32),
            pltpu.VMEM((C,), jnp.float32),
            pltpu.VMEM((NEG, C), jnp.float32),
            pltpu.SemaphoreType.DMA,
            pltpu.SemaphoreType.DMA,
        ],
    )(ctx_idx, lens, tar, neg, char_idx, cn, emb0, emb1, emb0c)


def _tc_loss_body(pos_ref, neg_ref, mask_ref, out_ref):
    p = jnp.clip(pos_ref[...], -10.0, 10.0)
    pos_loss = jnp.sum(jnp.log(1.0 + jnp.exp(-p)))
    m = mask_ref[...]
    q = jnp.clip(neg_ref[...] * m, -10.0, 10.0)
    neg_loss = jnp.sum(jnp.log(1.0 + jnp.exp(q)) * m)
    out_ref[0, 0] = pos_loss + neg_loss


def _tc_loss(pos2d, neg2d, mask2d):
    return pl.pallas_call(
        _tc_loss_body,
        out_shape=jax.ShapeDtypeStruct((1, 1), jnp.float32),
        out_specs=pl.BlockSpec(memory_space=pltpu.SMEM),
    )(pos2d, neg2d, mask2d)


def kernel(word_data, char_data, emb0_w, emb1_w, emb0_char_w):
    wd = word_data.astype(jnp.int32)
    cd = char_data.astype(jnp.int32)
    ctx_idx = wd[:, 0:NCTX].T
    lens = wd[:, NCTX].astype(jnp.float32)
    tar = wd[:, NCTX + 1]
    neg = wd[:, NCTX + 2:NCTX + 2 + NEG].reshape(-1)
    mask = wd[:, NCTX + 2 + NEG:].astype(jnp.float32)
    char_idx = cd[:, 0:NCHR].T
    cn = cd[:, NCHR].astype(jnp.float32)

    pos_ips, neg_ips = _sc_ips(ctx_idx, lens, tar, neg, char_idx, cn,
                               emb0_w, emb1_w, emb0_char_w)
    loss = _tc_loss(pos_ips.reshape(128, 128),
                    neg_ips.reshape(B * NEG // 128, 128),
                    mask.T.reshape(B * NEG // 128, 128))
    return loss[0, 0]
